# baseline (speedup 1.0000x reference)
import dataclasses

import numpy as np

import jax
import jax.numpy as jnp
from jax import lax
from jax.experimental import pallas as pl
from jax.experimental.pallas import tpu as pltpu
from jax.experimental.pallas import tpu_sc as plsc

N = 10000
E = 320000
XD = 128
PD = 3
ED = 16
H = 256
HH = 128
HW = 64
L = 4

NSUB = 16
CHUNK = 72
GCH = 6
NGRP = 48
NCH = GCH * NGRP
EPW = NCH * CHUNK
E_PAD = EPW * NSUB
DUMMY = N
AGG_ROWS = 10080
NZCH = AGG_ROWS // CHUNK

ROW_BLK = 400
EB_BLK = 512

_R128 = np.empty((HH,), np.int32)
for _b in range(HH // 32):
    for _i in range(16):
        _R128[32 * _b + 2 * _i] = 32 * _b + _i
        _R128[32 * _b + 2 * _i + 1] = 32 * _b + 16 + _i
_RIFFLE = np.concatenate([_R128, _R128 + HH])


def _node_enc_body(f_ref, w_ref, b_ref, o_ref):
    o_ref[0] = (
        jnp.dot(f_ref[...], w_ref[...], preferred_element_type=jnp.float32)
        + b_ref[...]
    )


def _encode_stack(feats, w, b, n_rows, row_blk, out_dtype):
    k = feats.shape[1]
    return pl.pallas_call(
        _node_enc_body if out_dtype == jnp.float32 else _edge_enc_body,
        grid=(2, n_rows // row_blk),
        in_specs=[
            pl.BlockSpec((row_blk, k), lambda i, j: (j, 0)),
            pl.BlockSpec((k, HH), lambda i, j: (0, i)),
            pl.BlockSpec((HH,), lambda i, j: (i,)),
        ],
        out_specs=pl.BlockSpec((1, row_blk, HH), lambda i, j: (i, j, 0)),
        out_shape=jax.ShapeDtypeStruct((2, n_rows, HH), out_dtype),
    )(feats, w, b)


def _edge_enc_body(f_ref, w_ref, b_ref, o_ref):
    o_ref[0] = (
        jnp.dot(f_ref[...], w_ref[...], preferred_element_type=jnp.float32)
        + b_ref[...]
    ).astype(jnp.bfloat16)


def _mlp_body(agg_ref, h_ref, w1_ref, b1_ref, w2_ref, b2_ref, o_ref):
    a = jnp.concatenate([agg_ref[0], agg_ref[1]], axis=1)
    t = jnp.maximum(
        jnp.dot(a, w1_ref[...], preferred_element_type=jnp.float32)
        + b1_ref[...],
        0.0,
    )
    o = jnp.dot(t, w2_ref[...], preferred_element_type=jnp.float32) + b2_ref[...]
    o_ref[0] = o[:, :HH] + h_ref[0]
    o_ref[1] = o[:, HH:] + h_ref[1]


def _mlp(agg, h, w1, b1, w2, b2):
    return pl.pallas_call(
        _mlp_body,
        grid=(N // ROW_BLK,),
        in_specs=[
            pl.BlockSpec((2, ROW_BLK, HH), lambda j: (0, j, 0)),
            pl.BlockSpec((2, ROW_BLK, HH), lambda j: (0, j, 0)),
            pl.BlockSpec((H, H), lambda j: (0, 0)),
            pl.BlockSpec((H,), lambda j: (0,)),
            pl.BlockSpec((H, H), lambda j: (0, 0)),
            pl.BlockSpec((H,), lambda j: (0,)),
        ],
        out_specs=pl.BlockSpec((2, ROW_BLK, HH), lambda j: (0, j, 0)),
        out_shape=jax.ShapeDtypeStruct((2, N, HH), jnp.float32),
    )(agg, h, w1, b1, w2, b2)


def _sc_body(h_hbm, ea_hbm, src_hbm, dst_hbm, out_hbm,
             srcv, dstv, hbuf, ebuf, aggs, gsem, esem, ssem, isem):
    c = lax.axis_index("c")
    s = lax.axis_index("s")

    zf = jnp.zeros((16,), jnp.float32)

    @pl.loop(0, CHUNK)
    def _zrow(i):
        for q in range(HH // 16):
            hbuf[0, i, pl.ds(q * 16, 16)] = zf

    @pl.loop(0, 9)
    def _zagg(k):
        zc = s + k * NSUB

        @pl.when(zc < NZCH)
        def _do():
            pltpu.sync_copy(hbuf.at[0], aggs.at[pl.ds(zc * CHUNK, CHUNK)])

    plsc.subcore_barrier()

    def wait_gather(p):
        pltpu.make_async_copy(
            h_hbm.at[c, pl.ds(0, CHUNK)], hbuf.at[p], gsem.at[p]).wait()

    def wait_ea(p):
        pltpu.make_async_copy(
            ea_hbm.at[c, pl.ds(0, CHUNK)], ebuf.at[p], esem.at[p]).wait()

    def wait_scatter(p):
        pltpu.make_async_copy(
            hbuf.at[p], aggs.at[pl.ds(0, CHUNK)], ssem.at[p]).wait()

    pltpu.sync_copy(src_hbm.at[s, 0], srcv.at[0])
    pltpu.sync_copy(dst_hbm.at[s, 0], dstv.at[0])
    pltpu.sync_copy(src_hbm.at[s, 1], srcv.at[1])
    pltpu.sync_copy(dst_hbm.at[s, 1], dstv.at[1])
    for b in range(2):
        pltpu.async_copy(
            h_hbm.at[c].at[srcv.at[0, b]], hbuf.at[b], gsem.at[b])
        pltpu.async_copy(
            ea_hbm.at[c, pl.ds(s * EPW + b * CHUNK, CHUNK)],
            ebuf.at[b], esem.at[b])

    @pl.loop(0, NGRP // 2)
    def _pair(t):
        for gg in range(2):
            g = 2 * t + gg
            for jj in range(GCH):
                k = g * GCH + jj
                p3 = jj % 3
                p2 = jj % 2
                q3 = (jj + 2) % 3

                wait_gather(p3)
                wait_ea(p2)

                @plsc.parallel_loop(0, CHUNK, unroll=2)
                def _row(i):
                    for q in range(HW // 16):
                        ew = plsc.bitcast(
                            ebuf[p2, i, pl.ds(q * 16, 16)], jnp.bfloat16)
                        ea_lo, ea_hi = plsc.unpack(
                            ew, format=plsc.PackFormat.INTERLEAVED,
                            preferred_element_type=jnp.float32)
                        s0 = pl.ds(q * 32, 16)
                        s1 = pl.ds(q * 32 + 16, 16)
                        hbuf[p3, i, s0] = jnp.maximum(
                            hbuf[p3, i, s0] + ea_lo, zf)
                        hbuf[p3, i, s1] = jnp.maximum(
                            hbuf[p3, i, s1] + ea_hi, zf)

                pltpu.async_copy(hbuf.at[p3], aggs.at[dstv.at[gg, jj]],
                                 ssem.at[p3], add=True)

                @pl.when(k + 2 < NCH)
                def _ea_next():
                    pltpu.async_copy(
                        ea_hbm.at[c, pl.ds(s * EPW + (k + 2) * CHUNK, CHUNK)],
                        ebuf.at[p2], esem.at[p2])

                @pl.when(k >= 1)
                def _ws():
                    wait_scatter(q3)

                if jj == 0:
                    @pl.when(g + 1 < NGRP)
                    def _ipf():
                        pltpu.async_copy(src_hbm.at[s, g + 1],
                                         srcv.at[1 - gg], isem.at[0])
                        pltpu.async_copy(dst_hbm.at[s, g + 1],
                                         dstv.at[1 - gg], isem.at[1])

                if jj == 4:
                    @pl.when(g + 1 < NGRP)
                    def _iw():
                        pltpu.make_async_copy(
                            src_hbm.at[s, 0], srcv.at[1 - gg],
                            isem.at[0]).wait()
                        pltpu.make_async_copy(
                            dst_hbm.at[s, 0], dstv.at[1 - gg],
                            isem.at[1]).wait()

                @pl.when(k + 2 < NCH)
                def _g_next():
                    if jj < 4:
                        sidx = srcv.at[gg, jj + 2]
                    else:
                        sidx = srcv.at[1 - gg, jj - 4]
                    pltpu.async_copy(h_hbm.at[c].at[sidx], hbuf.at[q3],
                                     gsem.at[q3])

    wait_scatter((NCH - 1) % 3)

    plsc.subcore_barrier()

    pltpu.sync_copy(aggs.at[pl.ds(s * 624, 624)],
                    out_hbm.at[c, pl.ds(s * 624, 624)])

    @pl.when(s == 0)
    def _tail():
        pltpu.sync_copy(aggs.at[pl.ds(9984, 16)],
                        out_hbm.at[c, pl.ds(9984, 16)])


def _sc_edge_phase(h_stack, ea_pk, src_r, dst_r):
    mesh = plsc.VectorSubcoreMesh(core_axis_name="c", subcore_axis_name="s")
    cp = pltpu.CompilerParams()
    if "needs_layout_passes" in pltpu.CompilerParams.__dataclass_fields__:
        cp = dataclasses.replace(cp, needs_layout_passes=False)
    kern = pl.kernel(
        _sc_body,
        out_type=jax.ShapeDtypeStruct((2, N, HH), jnp.float32),
        mesh=mesh,
        compiler_params=cp,
        scratch_types=[
            pltpu.VMEM((2, GCH, CHUNK), jnp.int32),
            pltpu.VMEM((2, GCH, CHUNK), jnp.int32),
            pltpu.VMEM((3, CHUNK, HH), jnp.float32),
            pltpu.VMEM((2, CHUNK, HW), jnp.int32),
            pltpu.VMEM_SHARED((AGG_ROWS, HH), jnp.float32),
            pltpu.SemaphoreType.DMA((3,)),
            pltpu.SemaphoreType.DMA((2,)),
            pltpu.SemaphoreType.DMA((3,)),
            pltpu.SemaphoreType.DMA((2,)),
        ],
    )
    return kern(h_stack, ea_pk, src_r, dst_r)


def kernel(x, pos, edge_attr, edge_index, batch, Wn, bn, We, be, W1, b1, W2, b2):
    del batch
    feats = jnp.concatenate(
        [x, pos, jnp.zeros((N, H - XD - PD), jnp.float32)], axis=1)
    wn_p = jnp.concatenate(
        [Wn, jnp.zeros((H - XD - PD, H), jnp.float32)], axis=0)
    ea_in = jnp.concatenate(
        [edge_attr, jnp.zeros((E_PAD - E, ED), jnp.float32)], axis=0)
    riffle = jnp.asarray(_RIFFLE)
    we_r = We[:, riffle]
    be_r = be[riffle]
    src = jnp.concatenate(
        [edge_index[0], jnp.zeros((E_PAD - E,), jnp.int32)]).reshape(
            NSUB, NGRP, GCH, CHUNK)
    dst = jnp.concatenate(
        [edge_index[1], jnp.full((E_PAD - E,), DUMMY, jnp.int32)]).reshape(
            NSUB, NGRP, GCH, CHUNK)

    h = _encode_stack(feats, wn_p, bn, N, ROW_BLK, jnp.float32)
    ea_bf = _encode_stack(ea_in, we_r, be_r, E_PAD, EB_BLK, jnp.bfloat16)
    ea_pk = lax.bitcast_convert_type(
        ea_bf.reshape(2, E_PAD, HW, 2), jnp.int32)

    for i in range(L):
        agg = _sc_edge_phase(h, ea_pk, src, dst)
        h = _mlp(agg, h, W1[i], b1[i], W2[i], b2[i])

    return jnp.concatenate([h[0], h[1]], axis=1)


# speedup vs baseline: 1.1622x; 1.1622x over previous
import jax
import jax.numpy as jnp
from jax import lax
from jax.experimental import pallas as pl
from jax.experimental.pallas import tpu as pltpu
from jax.experimental.pallas import tpu_sc as plsc

N = 10000
E = 320000
XD = 128
PD = 3
ED = 16
H = 256
HH = 128
L = 4

NSUB = 16
CHUNK = 64
GCH = 6
NGRP = 54
NCH = GCH * NGRP
EPW = NCH * CHUNK
E_PAD = EPW * NSUB
DUMMY = N
AGG_ROWS = 10240
NZCH = AGG_ROWS // CHUNK

ROW_BLK = 400
EB_BLK = 512


def _enc_body(f_ref, w_ref, b_ref, o_ref):
    o_ref[0] = (
        jnp.dot(f_ref[...], w_ref[...], preferred_element_type=jnp.float32)
        + b_ref[...]
    )


def _encode_stack(feats, w, b, n_rows, row_blk):
    k = feats.shape[1]
    return pl.pallas_call(
        _enc_body,
        grid=(2, n_rows // row_blk),
        in_specs=[
            pl.BlockSpec((row_blk, k), lambda i, j: (j, 0)),
            pl.BlockSpec((k, HH), lambda i, j: (0, i)),
            pl.BlockSpec((HH,), lambda i, j: (i,)),
        ],
        out_specs=pl.BlockSpec((1, row_blk, HH), lambda i, j: (i, j, 0)),
        out_shape=jax.ShapeDtypeStruct((2, n_rows, HH), jnp.float32),
    )(feats, w, b)


def _mlp_body(agg_ref, h_ref, w1_ref, b1_ref, w2_ref, b2_ref, o_ref):
    a = jnp.concatenate([agg_ref[0], agg_ref[1]], axis=1)
    t = jnp.maximum(
        jnp.dot(a, w1_ref[...], preferred_element_type=jnp.float32)
        + b1_ref[...],
        0.0,
    )
    o = jnp.dot(t, w2_ref[...], preferred_element_type=jnp.float32) + b2_ref[...]
    o_ref[0] = o[:, :HH] + h_ref[0]
    o_ref[1] = o[:, HH:] + h_ref[1]


def _mlp(agg, h, w1, b1, w2, b2):
    return pl.pallas_call(
        _mlp_body,
        grid=(N // ROW_BLK,),
        in_specs=[
            pl.BlockSpec((2, ROW_BLK, HH), lambda j: (0, j, 0)),
            pl.BlockSpec((2, ROW_BLK, HH), lambda j: (0, j, 0)),
            pl.BlockSpec((H, H), lambda j: (0, 0)),
            pl.BlockSpec((H,), lambda j: (0,)),
            pl.BlockSpec((H, H), lambda j: (0, 0)),
            pl.BlockSpec((H,), lambda j: (0,)),
        ],
        out_specs=pl.BlockSpec((2, ROW_BLK, HH), lambda j: (0, j, 0)),
        out_shape=jax.ShapeDtypeStruct((2, N, HH), jnp.float32),
    )(agg, h, w1, b1, w2, b2)


def _sc_body(h_hbm, ea_hbm, src_hbm, dst_hbm, out_hbm,
             srcv, dstv, hbuf, ebuf, aggs, gsem, esem, ssem, isem):
    c = lax.axis_index("c")
    s = lax.axis_index("s")

    zf = jnp.zeros((16,), jnp.float32)

    @pl.loop(0, CHUNK)
    def _zrow(i):
        for q in range(HH // 16):
            hbuf[0, i, pl.ds(q * 16, 16)] = zf

    @pl.loop(0, 10)
    def _zagg(k):
        pltpu.sync_copy(hbuf.at[0],
                        aggs.at[pl.ds((s + k * NSUB) * CHUNK, CHUNK)])

    plsc.subcore_barrier()

    def wait_gather(p):
        pltpu.make_async_copy(
            h_hbm.at[c, pl.ds(0, CHUNK)], hbuf.at[p], gsem.at[p]).wait()

    def wait_ea(p):
        pltpu.make_async_copy(
            ea_hbm.at[c, pl.ds(0, CHUNK)], ebuf.at[p], esem.at[p]).wait()

    def wait_scatter(p):
        pltpu.make_async_copy(
            hbuf.at[p], aggs.at[pl.ds(0, CHUNK)], ssem.at[p]).wait()

    pltpu.sync_copy(src_hbm.at[s, 0], srcv.at[0])
    pltpu.sync_copy(dst_hbm.at[s, 0], dstv.at[0])
    pltpu.sync_copy(src_hbm.at[s, 1], srcv.at[1])
    pltpu.sync_copy(dst_hbm.at[s, 1], dstv.at[1])
    for b in range(2):
        pltpu.async_copy(
            h_hbm.at[c].at[srcv.at[0, b]], hbuf.at[b], gsem.at[b])
        pltpu.async_copy(
            ea_hbm.at[c, pl.ds(s * EPW + b * CHUNK, CHUNK)],
            ebuf.at[b], esem.at[b])

    @pl.loop(0, NGRP // 2)
    def _pair(t):
        for gg in range(2):
            g = 2 * t + gg
            for jj in range(GCH):
                k = g * GCH + jj
                p3 = jj % 3
                p2 = jj % 2
                q3 = (jj + 2) % 3

                wait_gather(p3)
                wait_ea(p2)

                @plsc.parallel_loop(0, CHUNK, unroll=2)
                def _row(i):
                    for q in range(HH // 16):
                        sl = pl.ds(q * 16, 16)
                        hbuf[p3, i, sl] = jnp.maximum(
                            hbuf[p3, i, sl] + ebuf[p2, i, sl], zf)

                pltpu.async_copy(hbuf.at[p3], aggs.at[dstv.at[gg, jj]],
                                 ssem.at[p3], add=True)

                @pl.when(k + 2 < NCH)
                def _ea_next():
                    pltpu.async_copy(
                        ea_hbm.at[c, pl.ds(s * EPW + (k + 2) * CHUNK, CHUNK)],
                        ebuf.at[p2], esem.at[p2])

                @pl.when(k >= 1)
                def _ws():
                    wait_scatter(q3)

                if jj == 0:
                    @pl.when(g + 1 < NGRP)
                    def _ipf():
                        pltpu.async_copy(src_hbm.at[s, g + 1],
                                         srcv.at[1 - gg], isem.at[0])
                        pltpu.async_copy(dst_hbm.at[s, g + 1],
                                         dstv.at[1 - gg], isem.at[1])

                if jj == 4:
                    @pl.when(g + 1 < NGRP)
                    def _iw():
                        pltpu.make_async_copy(
                            src_hbm.at[s, 0], srcv.at[1 - gg],
                            isem.at[0]).wait()
                        pltpu.make_async_copy(
                            dst_hbm.at[s, 0], dstv.at[1 - gg],
                            isem.at[1]).wait()

                @pl.when(k + 2 < NCH)
                def _g_next():
                    if jj < 4:
                        sidx = srcv.at[gg, jj + 2]
                    else:
                        sidx = srcv.at[1 - gg, jj - 4]
                    pltpu.async_copy(h_hbm.at[c].at[sidx], hbuf.at[q3],
                                     gsem.at[q3])

    wait_scatter((NCH - 1) % 3)

    plsc.subcore_barrier()

    pltpu.sync_copy(aggs.at[pl.ds(s * 624, 624)],
                    out_hbm.at[c, pl.ds(s * 624, 624)])

    @pl.when(s == 0)
    def _tail():
        pltpu.sync_copy(aggs.at[pl.ds(9984, 16)],
                        out_hbm.at[c, pl.ds(9984, 16)])


def _sc_edge_phase(h_stack, ea_pk, src_r, dst_r):
    mesh = plsc.VectorSubcoreMesh(core_axis_name="c", subcore_axis_name="s")
    kern = pl.kernel(
        _sc_body,
        out_type=jax.ShapeDtypeStruct((2, N, HH), jnp.float32),
        mesh=mesh,
        scratch_types=[
            pltpu.VMEM((2, GCH, CHUNK), jnp.int32),
            pltpu.VMEM((2, GCH, CHUNK), jnp.int32),
            pltpu.VMEM((3, CHUNK, HH), jnp.float32),
            pltpu.VMEM((2, CHUNK, HH), jnp.float32),
            pltpu.VMEM_SHARED((AGG_ROWS, HH), jnp.float32),
            pltpu.SemaphoreType.DMA((3,)),
            pltpu.SemaphoreType.DMA((2,)),
            pltpu.SemaphoreType.DMA((3,)),
            pltpu.SemaphoreType.DMA((2,)),
        ],
    )
    return kern(h_stack, ea_pk, src_r, dst_r)


def kernel(x, pos, edge_attr, edge_index, batch, Wn, bn, We, be, W1, b1, W2, b2):
    del batch
    feats = jnp.concatenate(
        [x, pos, jnp.zeros((N, H - XD - PD), jnp.float32)], axis=1)
    wn_p = jnp.concatenate(
        [Wn, jnp.zeros((H - XD - PD, H), jnp.float32)], axis=0)
    ea_in = jnp.concatenate(
        [edge_attr, jnp.zeros((E_PAD - E, ED), jnp.float32)], axis=0)
    src = jnp.concatenate(
        [edge_index[0], jnp.zeros((E_PAD - E,), jnp.int32)]).reshape(
            NSUB, NGRP, GCH, CHUNK)
    dst = jnp.concatenate(
        [edge_index[1], jnp.full((E_PAD - E,), DUMMY, jnp.int32)]).reshape(
            NSUB, NGRP, GCH, CHUNK)

    h = _encode_stack(feats, wn_p, bn, N, ROW_BLK)
    ea = _encode_stack(ea_in, We, be, E_PAD, EB_BLK)

    for i in range(L):
        agg = _sc_edge_phase(h, ea, src, dst)
        h = _mlp(agg, h, W1[i], b1[i], W2[i], b2[i])

    return jnp.concatenate([h[0], h[1]], axis=1)
